# baseline (speedup 1.0000x reference)
import functools

import jax
import jax.numpy as jnp
from jax import lax
from jax.experimental import pallas as pl
from jax.experimental.pallas import tpu as pltpu
from jax.experimental.pallas import tpu_sc as plsc

_FEATURE_LAYERS = 5
_B = 128
_C = 8192
_E = 3000
_NL = 2 * _C
_K = 10
_G = 2
_NCAND = _G * _K

_NC, _NS, _L = 2, 16, 16
_NW = _NC * _NS
_ROWS_PER_W = _B // _NW

_CHUNKS_PER_GRP = 16
_GRPS = _C // (_L * _CHUNKS_PER_GRP)


def _mm_bias_body(x_ref, w_ref, b_ref, o_ref):
    o_ref[...] = (
        jnp.dot(x_ref[...], w_ref[...], preferred_element_type=jnp.float32)
        + b_ref[...]
    )


def _matmul_bias(x, w, b, block_n):
    m, k = x.shape
    n = w.shape[1]
    return pl.pallas_call(
        _mm_bias_body,
        grid=(pl.cdiv(n, block_n),),
        in_specs=[
            pl.BlockSpec((m, k), lambda j: (0, 0)),
            pl.BlockSpec((k, block_n), lambda j: (0, j)),
            pl.BlockSpec((1, block_n), lambda j: (0, j)),
        ],
        out_specs=pl.BlockSpec((m, block_n), lambda j: (0, j)),
        out_shape=jax.ShapeDtypeStruct((m, n), jnp.float32),
    )(x, w, b.reshape(1, n))


def _mmT_bias_body(w_ref, x_ref, b_ref, o_ref):
    o_ref[...] = (
        lax.dot_general(w_ref[...], x_ref[...], (((1,), (1,)), ((), ())),
                        preferred_element_type=jnp.float32)
        + b_ref[...]
    )


def _matmulT_bias(wT, x, b, block_m, n_pad=None):
    n, k = wT.shape
    m = x.shape[0]
    n_out = n if n_pad is None else n_pad
    return pl.pallas_call(
        _mmT_bias_body,
        grid=(pl.cdiv(n_out, block_m),),
        in_specs=[
            pl.BlockSpec((block_m, k), lambda j: (j, 0)),
            pl.BlockSpec((m, k), lambda j: (0, 0)),
            pl.BlockSpec((block_m, 1), lambda j: (j, 0)),
        ],
        out_specs=pl.BlockSpec((block_m, m), lambda j: (j, 0)),
        out_shape=jax.ShapeDtypeStruct((n_out, m), jnp.float32),
    )(wT, x, b.reshape(n, 1))


_NCHUNK = _B * _NCAND // _L
_D_PER = 94
_D_PAD = _NW * _D_PER


def _sc_gather_dot_body(cand_hbm, tableP_hbm, embT_hbm, out_hbm,
                        cand_v, bidx_v, embT_v, row_v, acc_v, sem0, sem1):
    wid = lax.axis_index("s") * _NC + lax.axis_index("c")
    lane = lax.iota(jnp.int32, _L)
    d0 = wid * _D_PER
    nd = jnp.minimum(_D_PER, _E - d0)
    npair = nd // 2

    pltpu.sync_copy(cand_hbm, cand_v)
    pltpu.sync_copy(embT_hbm.at[pl.ds(d0, _D_PER)], embT_v)

    def init_bidx(i, _):
        bidx_v[pl.ds(i * _L, _L)] = (i * _L + lane) // _NCAND
        return 0
    lax.fori_loop(0, _NCHUNK, init_bidx, 0)
    zero = jnp.zeros((_L,), jnp.float32)
    for i in range(_NCHUNK):
        acc_v[0, i, :] = zero

    def start_pair(p, parity):
        src = tableP_hbm.at[pl.ds(d0 + 2 * p, 2)]

        @pl.when(parity == 0)
        def _():
            pltpu.async_copy(src, row_v.at[pl.ds(0, 2)], sem0)

        @pl.when(parity != 0)
        def _():
            pltpu.async_copy(src, row_v.at[pl.ds(2, 2)], sem1)

    def wait_pair(parity):
        dummy = tableP_hbm.at[pl.ds(0, 2)]

        @pl.when(parity == 0)
        def _():
            pltpu.make_async_copy(dummy, row_v.at[pl.ds(0, 2)], sem0).wait()

        @pl.when(parity != 0)
        def _():
            pltpu.make_async_copy(dummy, row_v.at[pl.ds(2, 2)], sem1).wait()

    @pl.when(npair > 0)
    def _():
        start_pair(jnp.int32(0), jnp.int32(0))

    def pair_body(p, _):
        parity = lax.rem(p, 2)

        @pl.when(p + 1 < npair)
        def _():
            start_pair(p + 1, 1 - parity)

        wait_pair(parity)
        for j in range(2):
            d_local = 2 * p + j
            dsplat = jnp.full((_L,), 0, jnp.int32) + d_local
            rsplat = jnp.full((_L,), 0, jnp.int32) + (parity * 2 + j)
            for i in range(_NCHUNK):
                cch = cand_v[pl.ds(i * _L, _L)]
                bch = bidx_v[pl.ds(i * _L, _L)]
                ech = plsc.load_gather(embT_v, [dsplat, bch])
                g = plsc.load_gather(row_v, [rsplat, cch])
                acc_v[0, i, :] = acc_v[0, i, :] + g * ech
        return 0

    lax.fori_loop(0, npair, pair_body, 0)
    pltpu.sync_copy(acc_v, out_hbm.at[pl.ds(wid, 1)])


_sc_gather_dot = functools.partial(
    pl.kernel,
    mesh=plsc.VectorSubcoreMesh(core_axis_name="c", subcore_axis_name="s"),
    out_type=jax.ShapeDtypeStruct((_NW, _NCHUNK, _L), jnp.float32),
    scratch_types=[
        pltpu.VMEM((_B * _NCAND,), jnp.int32),
        pltpu.VMEM((_B * _NCAND,), jnp.int32),
        pltpu.VMEM((_D_PER, _B), jnp.float32),
        pltpu.VMEM((4, _NL), jnp.float32),
        pltpu.VMEM((1, _NCHUNK, _L), jnp.float32),
        pltpu.SemaphoreType.DMA,
        pltpu.SemaphoreType.DMA,
    ],
    compiler_params=pltpu.CompilerParams(
        needs_layout_passes=False, use_tc_tiling_on_sc=False),
)(_sc_gather_dot_body)


def _sc_body(logits_hbm, gy_hbm, cand_hbm, row_v, gy_v, cand_v):
    wid = lax.axis_index("s") * _NC + lax.axis_index("c")
    base = wid * _ROWS_PER_W
    lane = lax.iota(jnp.int32, _L)
    k_mask = lane < _K
    neg_inf = jnp.full((_L,), -jnp.inf, jnp.float32)

    pltpu.sync_copy(gy_hbm, gy_v)

    for r in range(_ROWS_PER_W):
        b = base + r
        pltpu.sync_copy(logits_hbm.at[pl.ds(b, 1)], row_v)

        def grp_body(g, carry):
            cval, cidx, thr = carry
            gbase = g * (_L * _CHUNKS_PER_GRP)
            m = row_v[0, pl.ds(gbase, _L)]
            for j in range(1, _CHUNKS_PER_GRP):
                m = jnp.maximum(m, row_v[0, pl.ds(gbase + j * _L, _L)])
            gmax = jnp.max(m)

            def merge(c3):
                cv, ci, _ = c3
                for j in range(_CHUNKS_PER_GRP):
                    v = row_v[0, pl.ds(gbase + j * _L, _L)]
                    vi = gbase + j * _L + lane
                    sv, si = plsc.sort_key_val(v, vi, descending=True)
                    rv = lax.rev(sv, (0,))
                    ri = lax.rev(si, (0,))
                    take = rv > cv
                    nv = jnp.where(take, rv, cv)
                    ni = jnp.where(take, ri, ci)
                    cv, ci = plsc.sort_key_val(nv, ni, descending=True)
                new_thr = jnp.min(jnp.where(k_mask, cv, jnp.inf))
                return cv, ci, new_thr

            return lax.cond(gmax > thr, merge, lambda c3: c3,
                            (cval, cidx, thr))

        _, cidx, _ = lax.fori_loop(
            0, _GRPS, grp_body,
            (neg_inf, jnp.zeros((_L,), jnp.int32), -jnp.inf))

        safe_idx = jnp.where(k_mask, cidx, 0)
        ev = plsc.load_gather(gy_v, [safe_idx * 2])
        ov = plsc.load_gather(gy_v, [safe_idx * 2 + 1])
        plsc.store_scatter(cand_v, [r * _NCAND + lane * 2], ev, mask=k_mask)
        plsc.store_scatter(cand_v, [r * _NCAND + lane * 2 + 1], ov,
                           mask=k_mask)

    pltpu.sync_copy(
        cand_v, cand_hbm.at[pl.ds(base * _NCAND, _ROWS_PER_W * _NCAND)])


_sc_topk_route = functools.partial(
    pl.kernel,
    mesh=plsc.VectorSubcoreMesh(core_axis_name="c", subcore_axis_name="s"),
    out_type=jax.ShapeDtypeStruct((_B * _NCAND,), jnp.int32),
    scratch_types=[
        pltpu.VMEM((1, _C), jnp.float32),
        pltpu.VMEM((_NL,), jnp.int32),
        pltpu.VMEM((_ROWS_PER_W * _NCAND,), jnp.int32),
    ],
    compiler_params=pltpu.CompilerParams(
        needs_layout_passes=False, use_tc_tiling_on_sc=False),
)(_sc_body)


@jax.jit
def kernel(hidden_states, labels, W1, b1, W2, b2, group_y, embed_table):
    del labels
    cls_feats = jnp.concatenate(
        [hidden_states[-i][:, 0] for i in range(1, _FEATURE_LAYERS + 1)],
        axis=-1)
    logits = _matmul_bias(cls_feats, W1, b1, 1024)
    cand = _sc_topk_route(logits, group_y.reshape(-1))
    embT = _matmulT_bias(W2.T, logits, b2, 512, n_pad=_D_PAD)
    partials = _sc_gather_dot(cand, embed_table.T, embT)
    return partials.reshape(_NW, _B * _NCAND).sum(axis=0).reshape(_B, _NCAND)


# speedup vs baseline: 1.7051x; 1.7051x over previous
import functools

import jax
import jax.numpy as jnp
from jax import lax
from jax.experimental import pallas as pl
from jax.experimental.pallas import tpu as pltpu
from jax.experimental.pallas import tpu_sc as plsc

_FEATURE_LAYERS = 5
_B = 128
_C = 8192
_E = 3000
_NL = 2 * _C
_K = 10
_G = 2
_NCAND = _G * _K

_NC, _NS, _L = 2, 16, 16
_NW = _NC * _NS
_ROWS_PER_W = _B // _NW

_CHUNKS_PER_GRP = 16
_GRPS = _C // (_L * _CHUNKS_PER_GRP)


def _mm_bias_body(x_ref, w_ref, b_ref, o_ref):
    o_ref[...] = (
        jnp.dot(x_ref[...], w_ref[...], preferred_element_type=jnp.float32)
        + b_ref[...]
    )


def _matmul_bias(x, w, b, block_n):
    m, k = x.shape
    n = w.shape[1]
    return pl.pallas_call(
        _mm_bias_body,
        grid=(pl.cdiv(n, block_n),),
        in_specs=[
            pl.BlockSpec((m, k), lambda j: (0, 0)),
            pl.BlockSpec((k, block_n), lambda j: (0, j)),
            pl.BlockSpec((1, block_n), lambda j: (0, j)),
        ],
        out_specs=pl.BlockSpec((m, block_n), lambda j: (0, j)),
        out_shape=jax.ShapeDtypeStruct((m, n), jnp.float32),
    )(x, w, b.reshape(1, n))


def _mmT_bias_body(w_ref, x_ref, b_ref, o_ref):
    o_ref[...] = (
        lax.dot_general(w_ref[...], x_ref[...], (((1,), (1,)), ((), ())),
                        preferred_element_type=jnp.float32)
        + b_ref[...]
    )


def _matmulT_bias(wT, x, b, block_m):
    n, k = wT.shape
    m = x.shape[0]
    return pl.pallas_call(
        _mmT_bias_body,
        grid=(pl.cdiv(n, block_m),),
        in_specs=[
            pl.BlockSpec((block_m, k), lambda j: (j, 0)),
            pl.BlockSpec((m, k), lambda j: (0, 0)),
            pl.BlockSpec((block_m, 1), lambda j: (j, 0)),
        ],
        out_specs=pl.BlockSpec((block_m, m), lambda j: (j, 0)),
        out_shape=jax.ShapeDtypeStruct((n, m), jnp.float32),
    )(wT, x, b.reshape(n, 1))


_TR, _TC_ = 512, 2048


def _transpose_body(x_ref, o_ref):
    o_ref[...] = x_ref[...].T


def _transpose(xT):
    n, m = xT.shape
    return pl.pallas_call(
        _transpose_body,
        grid=(pl.cdiv(n, _TR), pl.cdiv(m, _TC_)),
        in_specs=[pl.BlockSpec((_TR, _TC_), lambda i, j: (i, j))],
        out_specs=pl.BlockSpec((_TC_, _TR), lambda i, j: (j, i)),
        out_shape=jax.ShapeDtypeStruct((m, n), jnp.float32),
    )(xT)


_RB = 8


def _gather_dot_body(cand_ref, table_ref, emb_ref, o_ref, rows_v, sem):
    g = pl.program_id(0)

    def _copy(j):
        idx = cand_ref[(g * _RB) * _NCAND + j]
        return pltpu.make_async_copy(
            table_ref.at[pl.ds(idx, 1)], rows_v.at[pl.ds(j, 1)], sem)

    for j in range(_RB * _NCAND):
        _copy(j).start()
    for j in range(_RB * _NCAND):
        _copy(j).wait()
    for i in range(_RB):
        o_ref[pl.ds(i, 1), :] = lax.dot_general(
            emb_ref[pl.ds(i, 1), :], rows_v[pl.ds(i * _NCAND, _NCAND), :],
            (((1,), (1,)), ((), ())),
            preferred_element_type=jnp.float32)


def _gather_dot(cand, table_rm, emb):
    return pl.pallas_call(
        _gather_dot_body,
        grid_spec=pltpu.PrefetchScalarGridSpec(
            num_scalar_prefetch=1,
            grid=(_B // _RB,),
            in_specs=[
                pl.BlockSpec(memory_space=pl.ANY),
                pl.BlockSpec((_RB, _E), lambda g, c: (g, 0)),
            ],
            out_specs=pl.BlockSpec((_RB, _NCAND), lambda g, c: (g, 0)),
            scratch_shapes=[
                pltpu.VMEM((_RB * _NCAND, _E), jnp.float32),
                pltpu.SemaphoreType.DMA,
            ],
        ),
        out_shape=jax.ShapeDtypeStruct((_B, _NCAND), jnp.float32),
    )(cand, table_rm, emb)


def _sc_body(logits_hbm, gy_hbm, cand_hbm, row_v, gy_v, cand_v):
    wid = lax.axis_index("s") * _NC + lax.axis_index("c")
    base = wid * _ROWS_PER_W
    lane = lax.iota(jnp.int32, _L)
    k_mask = lane < _K
    neg_inf = jnp.full((_L,), -jnp.inf, jnp.float32)

    pltpu.sync_copy(gy_hbm, gy_v)

    for r in range(_ROWS_PER_W):
        b = base + r
        pltpu.sync_copy(logits_hbm.at[pl.ds(b, 1)], row_v)

        def grp_body(g, carry):
            cval, cidx, thr = carry
            gbase = g * (_L * _CHUNKS_PER_GRP)
            m = row_v[0, pl.ds(gbase, _L)]
            for j in range(1, _CHUNKS_PER_GRP):
                m = jnp.maximum(m, row_v[0, pl.ds(gbase + j * _L, _L)])
            gmax = jnp.max(m)

            def merge(c3):
                cv, ci, _ = c3
                for j in range(_CHUNKS_PER_GRP):
                    v = row_v[0, pl.ds(gbase + j * _L, _L)]
                    vi = gbase + j * _L + lane
                    sv, si = plsc.sort_key_val(v, vi, descending=True)
                    rv = lax.rev(sv, (0,))
                    ri = lax.rev(si, (0,))
                    take = rv > cv
                    nv = jnp.where(take, rv, cv)
                    ni = jnp.where(take, ri, ci)
                    cv, ci = plsc.sort_key_val(nv, ni, descending=True)
                new_thr = jnp.min(jnp.where(k_mask, cv, jnp.inf))
                return cv, ci, new_thr

            return lax.cond(gmax > thr, merge, lambda c3: c3,
                            (cval, cidx, thr))

        _, cidx, _ = lax.fori_loop(
            0, _GRPS, grp_body,
            (neg_inf, jnp.zeros((_L,), jnp.int32), -jnp.inf))

        safe_idx = jnp.where(k_mask, cidx, 0)
        ev = plsc.load_gather(gy_v, [safe_idx * 2])
        ov = plsc.load_gather(gy_v, [safe_idx * 2 + 1])
        plsc.store_scatter(cand_v, [r * _NCAND + lane * 2], ev, mask=k_mask)
        plsc.store_scatter(cand_v, [r * _NCAND + lane * 2 + 1], ov,
                           mask=k_mask)

    pltpu.sync_copy(
        cand_v, cand_hbm.at[pl.ds(base * _NCAND, _ROWS_PER_W * _NCAND)])


_sc_topk_route = functools.partial(
    pl.kernel,
    mesh=plsc.VectorSubcoreMesh(core_axis_name="c", subcore_axis_name="s"),
    out_type=jax.ShapeDtypeStruct((_B * _NCAND,), jnp.int32),
    scratch_types=[
        pltpu.VMEM((1, _C), jnp.float32),
        pltpu.VMEM((_NL,), jnp.int32),
        pltpu.VMEM((_ROWS_PER_W * _NCAND,), jnp.int32),
    ],
    compiler_params=pltpu.CompilerParams(
        needs_layout_passes=False, use_tc_tiling_on_sc=False),
)(_sc_body)


@jax.jit
def kernel(hidden_states, labels, W1, b1, W2, b2, group_y, embed_table):
    del labels
    cls_feats = jnp.concatenate(
        [hidden_states[-i][:, 0] for i in range(1, _FEATURE_LAYERS + 1)],
        axis=-1)
    logits = _matmul_bias(cls_feats, W1, b1, 1024)
    cand = _sc_topk_route(logits, group_y.reshape(-1))
    embT = _matmulT_bias(W2.T, logits, b2, 512)
    table_rm = _transpose(embed_table.T)
    return _gather_dot(cand, table_rm, embT.T)


# speedup vs baseline: 2.1667x; 1.2707x over previous
import functools

import jax
import jax.numpy as jnp
from jax import lax
from jax.experimental import pallas as pl
from jax.experimental.pallas import tpu as pltpu
from jax.experimental.pallas import tpu_sc as plsc

_FEATURE_LAYERS = 5
_B = 128
_C = 8192
_E = 3000
_NL = 2 * _C
_K = 10
_G = 2
_NCAND = _G * _K

_NC, _NS, _L = 2, 16, 16
_NW = _NC * _NS
_ROWS_PER_W = _B // _NW

_CHUNKS_PER_GRP = 16
_GRPS = _C // (_L * _CHUNKS_PER_GRP)


def _mm_bias_body(x_ref, w_ref, b_ref, o_ref):
    o_ref[...] = (
        jnp.dot(x_ref[...], w_ref[...], preferred_element_type=jnp.float32)
        + b_ref[...]
    )


def _matmul_bias(x, w, b, block_n):
    m, k = x.shape
    n = w.shape[1]
    return pl.pallas_call(
        _mm_bias_body,
        grid=(pl.cdiv(n, block_n),),
        in_specs=[
            pl.BlockSpec((m, k), lambda j: (0, 0)),
            pl.BlockSpec((k, block_n), lambda j: (0, j)),
            pl.BlockSpec((1, block_n), lambda j: (0, j)),
        ],
        out_specs=pl.BlockSpec((m, block_n), lambda j: (0, j)),
        out_shape=jax.ShapeDtypeStruct((m, n), jnp.float32),
    )(x, w, b.reshape(1, n))


def _mmT_bias_body(w_ref, x_ref, b_ref, o_ref):
    o_ref[...] = (
        lax.dot_general(w_ref[...], x_ref[...], (((1,), (1,)), ((), ())),
                        preferred_element_type=jnp.float32)
        + b_ref[...]
    )


def _matmulT_bias(wT, x, b, block_m):
    n, k = wT.shape
    m = x.shape[0]
    return pl.pallas_call(
        _mmT_bias_body,
        grid=(pl.cdiv(n, block_m),),
        in_specs=[
            pl.BlockSpec((block_m, k), lambda j: (j, 0)),
            pl.BlockSpec((m, k), lambda j: (0, 0)),
            pl.BlockSpec((block_m, 1), lambda j: (j, 0)),
        ],
        out_specs=pl.BlockSpec((block_m, m), lambda j: (j, 0)),
        out_shape=jax.ShapeDtypeStruct((n, m), jnp.float32),
    )(wT, x, b.reshape(n, 1))


def _score_body(e_ref, t_ref, o_ref):
    o_ref[...] = jnp.dot(e_ref[...], t_ref[...],
                         preferred_element_type=jnp.float32)


def _score_all(emb, tableP):
    m, k = emb.shape
    n = tableP.shape[1]
    bn = 1024
    return pl.pallas_call(
        _score_body,
        grid=(n // bn,),
        in_specs=[
            pl.BlockSpec((m, k), lambda j: (0, 0)),
            pl.BlockSpec((k, bn), lambda j: (0, j)),
        ],
        out_specs=pl.BlockSpec((m, bn), lambda j: (0, j)),
        out_shape=jax.ShapeDtypeStruct((m, n), jnp.float32),
    )(emb, tableP)


def _sc_body(logits_hbm, gy_hbm, cand_hbm, row_v, gy_v, cand_v):
    wid = lax.axis_index("s") * _NC + lax.axis_index("c")
    base = wid * _ROWS_PER_W
    lane = lax.iota(jnp.int32, _L)
    k_mask = lane < _K
    neg_inf = jnp.full((_L,), -jnp.inf, jnp.float32)

    pltpu.sync_copy(gy_hbm, gy_v)

    for r in range(_ROWS_PER_W):
        b = base + r
        pltpu.sync_copy(logits_hbm.at[pl.ds(b, 1)], row_v)

        def grp_body(g, carry):
            cval, cidx, thr = carry
            gbase = g * (_L * _CHUNKS_PER_GRP)
            m = row_v[0, pl.ds(gbase, _L)]
            for j in range(1, _CHUNKS_PER_GRP):
                m = jnp.maximum(m, row_v[0, pl.ds(gbase + j * _L, _L)])
            gmax = jnp.max(m)

            def merge(c3):
                cv, ci, _ = c3
                for j in range(_CHUNKS_PER_GRP):
                    v = row_v[0, pl.ds(gbase + j * _L, _L)]
                    vi = gbase + j * _L + lane
                    sv, si = plsc.sort_key_val(v, vi, descending=True)
                    rv = lax.rev(sv, (0,))
                    ri = lax.rev(si, (0,))
                    take = rv > cv
                    nv = jnp.where(take, rv, cv)
                    ni = jnp.where(take, ri, ci)
                    cv, ci = plsc.sort_key_val(nv, ni, descending=True)
                new_thr = jnp.min(jnp.where(k_mask, cv, jnp.inf))
                return cv, ci, new_thr

            return lax.cond(gmax > thr, merge, lambda c3: c3,
                            (cval, cidx, thr))

        _, cidx, _ = lax.fori_loop(
            0, _GRPS, grp_body,
            (neg_inf, jnp.zeros((_L,), jnp.int32), -jnp.inf))

        safe_idx = jnp.where(k_mask, cidx, 0)
        ev = plsc.load_gather(gy_v, [safe_idx * 2])
        ov = plsc.load_gather(gy_v, [safe_idx * 2 + 1])
        plsc.store_scatter(cand_v, [r * _NCAND + lane * 2], ev, mask=k_mask)
        plsc.store_scatter(cand_v, [r * _NCAND + lane * 2 + 1], ov,
                           mask=k_mask)

    pltpu.sync_copy(
        cand_v, cand_hbm.at[pl.ds(base * _NCAND, _ROWS_PER_W * _NCAND)])


_sc_topk_route = functools.partial(
    pl.kernel,
    mesh=plsc.VectorSubcoreMesh(core_axis_name="c", subcore_axis_name="s"),
    out_type=jax.ShapeDtypeStruct((_B * _NCAND,), jnp.int32),
    scratch_types=[
        pltpu.VMEM((1, _C), jnp.float32),
        pltpu.VMEM((_NL,), jnp.int32),
        pltpu.VMEM((_ROWS_PER_W * _NCAND,), jnp.int32),
    ],
    compiler_params=pltpu.CompilerParams(
        needs_layout_passes=False, use_tc_tiling_on_sc=False),
)(_sc_body)


def _sc_select_body(s_hbm, cand_hbm, out_hbm, s4_v, cand_v, out_v):
    wid = lax.axis_index("s") * _NC + lax.axis_index("c")
    base = wid * _ROWS_PER_W
    nc = _ROWS_PER_W * _NCAND
    lane = lax.iota(jnp.int32, _L)
    pltpu.sync_copy(s_hbm.at[pl.ds(base, _ROWS_PER_W)], s4_v)
    pltpu.sync_copy(cand_hbm.at[pl.ds(base * _NCAND, nc)], cand_v)
    for i in range(nc // _L):
        rowv = (i * _L + lane) // _NCAND
        colv = cand_v[pl.ds(i * _L, _L)]
        out_v[pl.ds(i * _L, _L)] = plsc.load_gather(s4_v, [rowv, colv])
    pltpu.sync_copy(out_v, out_hbm.at[pl.ds(base * _NCAND, nc)])


_sc_select = functools.partial(
    pl.kernel,
    mesh=plsc.VectorSubcoreMesh(core_axis_name="c", subcore_axis_name="s"),
    out_type=jax.ShapeDtypeStruct((_B * _NCAND,), jnp.float32),
    scratch_types=[
        pltpu.VMEM((_ROWS_PER_W, _NL), jnp.float32),
        pltpu.VMEM((_ROWS_PER_W * _NCAND,), jnp.int32),
        pltpu.VMEM((_ROWS_PER_W * _NCAND,), jnp.float32),
    ],
    compiler_params=pltpu.CompilerParams(
        needs_layout_passes=False, use_tc_tiling_on_sc=False),
)(_sc_select_body)


@jax.jit
def kernel(hidden_states, labels, W1, b1, W2, b2, group_y, embed_table):
    del labels
    cls_feats = jnp.concatenate(
        [hidden_states[-i][:, 0] for i in range(1, _FEATURE_LAYERS + 1)],
        axis=-1)
    logits = _matmul_bias(cls_feats, W1, b1, 1024)
    cand = _sc_topk_route(logits, group_y.reshape(-1))
    embT = _matmulT_bias(W2.T, logits, b2, 512)
    scores = _score_all(embT.T, embed_table.T)
    return _sc_select(scores, cand).reshape(_B, _NCAND)


# speedup vs baseline: 2.1693x; 1.0012x over previous
import functools

import jax
import jax.numpy as jnp
from jax import lax
from jax.experimental import pallas as pl
from jax.experimental.pallas import tpu as pltpu
from jax.experimental.pallas import tpu_sc as plsc

_FEATURE_LAYERS = 5
_B = 128
_C = 8192
_E = 3000
_NL = 2 * _C
_K = 10
_G = 2
_NCAND = _G * _K

_NC, _NS, _L = 2, 16, 16
_NW = _NC * _NS
_ROWS_PER_W = _B // _NW

_CHUNKS_PER_GRP = 16
_GRPS = _C // (_L * _CHUNKS_PER_GRP)


def _mm_bias_body(x_ref, w_ref, b_ref, o_ref):
    o_ref[...] = (
        jnp.dot(x_ref[...], w_ref[...], preferred_element_type=jnp.float32)
        + b_ref[...]
    )


def _matmul_bias(x, w, b, block_n):
    m, k = x.shape
    n = w.shape[1]
    return pl.pallas_call(
        _mm_bias_body,
        grid=(pl.cdiv(n, block_n),),
        in_specs=[
            pl.BlockSpec((m, k), lambda j: (0, 0)),
            pl.BlockSpec((k, block_n), lambda j: (0, j)),
            pl.BlockSpec((1, block_n), lambda j: (0, j)),
        ],
        out_specs=pl.BlockSpec((m, block_n), lambda j: (0, j)),
        out_shape=jax.ShapeDtypeStruct((m, n), jnp.float32),
    )(x, w, b.reshape(1, n))


def _mmT_bias_body(w_ref, x_ref, b_ref, o_ref):
    o_ref[...] = (
        lax.dot_general(w_ref[...], x_ref[...], (((1,), (1,)), ((), ())),
                        preferred_element_type=jnp.float32)
        + b_ref[...]
    )


def _matmulT_bias(wT, x, b, block_m):
    n, k = wT.shape
    m = x.shape[0]
    return pl.pallas_call(
        _mmT_bias_body,
        grid=(pl.cdiv(n, block_m),),
        in_specs=[
            pl.BlockSpec((block_m, k), lambda j: (j, 0)),
            pl.BlockSpec((m, k), lambda j: (0, 0)),
            pl.BlockSpec((block_m, 1), lambda j: (j, 0)),
        ],
        out_specs=pl.BlockSpec((block_m, m), lambda j: (j, 0)),
        out_shape=jax.ShapeDtypeStruct((n, m), jnp.float32),
    )(wT, x, b.reshape(n, 1))


def _score_body(e_ref, t_ref, o_ref):
    o_ref[...] = jnp.dot(e_ref[...], t_ref[...],
                         preferred_element_type=jnp.float32)


def _score_all(emb, tableP):
    m, k = emb.shape
    n = tableP.shape[1]
    bn = 1024
    return pl.pallas_call(
        _score_body,
        grid=(n // bn,),
        in_specs=[
            pl.BlockSpec((m, k), lambda j: (0, 0)),
            pl.BlockSpec((k, bn), lambda j: (0, j)),
        ],
        out_specs=pl.BlockSpec((m, bn), lambda j: (0, j)),
        out_shape=jax.ShapeDtypeStruct((m, n), jnp.float32),
    )(emb, tableP)


def _sc_body(logits_hbm, gy_hbm, cand_hbm, row_v, gy_v, cand_v):
    wid = lax.axis_index("s") * _NC + lax.axis_index("c")
    base = wid * _ROWS_PER_W
    lane = lax.iota(jnp.int32, _L)
    k_mask = lane < _K
    neg_inf = jnp.full((_L,), -jnp.inf, jnp.float32)

    pltpu.sync_copy(gy_hbm, gy_v)

    for r in range(_ROWS_PER_W):
        b = base + r
        pltpu.sync_copy(logits_hbm.at[pl.ds(b, 1)], row_v)

        def grp_body(g, carry):
            cval, cidx, thr = carry
            gbase = g * (_L * _CHUNKS_PER_GRP)
            m = row_v[0, pl.ds(gbase, _L)]
            for j in range(1, _CHUNKS_PER_GRP):
                m = jnp.maximum(m, row_v[0, pl.ds(gbase + j * _L, _L)])
            gmax = jnp.max(m)

            def merge(c3):
                cv, ci, _ = c3
                for j in range(_CHUNKS_PER_GRP):
                    v = row_v[0, pl.ds(gbase + j * _L, _L)]
                    vi = gbase + j * _L + lane
                    sv, si = plsc.sort_key_val(v, vi, descending=True)
                    rv = lax.rev(sv, (0,))
                    ri = lax.rev(si, (0,))
                    take = rv > cv
                    nv = jnp.where(take, rv, cv)
                    ni = jnp.where(take, ri, ci)
                    cv, ci = plsc.sort_key_val(nv, ni, descending=True)
                new_thr = jnp.min(jnp.where(k_mask, cv, jnp.inf))
                return cv, ci, new_thr

            return lax.cond(gmax > thr, merge, lambda c3: c3,
                            (cval, cidx, thr))

        _, cidx, _ = lax.fori_loop(
            0, _GRPS, grp_body,
            (neg_inf, jnp.zeros((_L,), jnp.int32), -jnp.inf))

        safe_idx = jnp.where(k_mask, cidx, 0)
        ev = plsc.load_gather(gy_v, [safe_idx * 2])
        ov = plsc.load_gather(gy_v, [safe_idx * 2 + 1])
        plsc.store_scatter(cand_v, [r * _NCAND + lane * 2], ev, mask=k_mask)
        plsc.store_scatter(cand_v, [r * _NCAND + lane * 2 + 1], ov,
                           mask=k_mask)

    pltpu.sync_copy(
        cand_v, cand_hbm.at[pl.ds(base * _NCAND, _ROWS_PER_W * _NCAND)])


_sc_topk_route = functools.partial(
    pl.kernel,
    mesh=plsc.VectorSubcoreMesh(core_axis_name="c", subcore_axis_name="s"),
    out_type=jax.ShapeDtypeStruct((_B * _NCAND,), jnp.int32),
    scratch_types=[
        pltpu.VMEM((1, _C), jnp.float32),
        pltpu.VMEM((_NL,), jnp.int32),
        pltpu.VMEM((_ROWS_PER_W * _NCAND,), jnp.int32),
    ],
    compiler_params=pltpu.CompilerParams(
        needs_layout_passes=False, use_tc_tiling_on_sc=False,
        has_side_effects=True),
)(_sc_body)


def _sc_select_body(s_hbm, cand_hbm, out_hbm, s4_v, cand_v, out_v):
    wid = lax.axis_index("s") * _NC + lax.axis_index("c")
    base = wid * _ROWS_PER_W
    nc = _ROWS_PER_W * _NCAND
    lane = lax.iota(jnp.int32, _L)
    pltpu.sync_copy(s_hbm.at[pl.ds(base, _ROWS_PER_W)], s4_v)
    pltpu.sync_copy(cand_hbm.at[pl.ds(base * _NCAND, nc)], cand_v)
    for i in range(nc // _L):
        rowv = (i * _L + lane) // _NCAND
        colv = cand_v[pl.ds(i * _L, _L)]
        out_v[pl.ds(i * _L, _L)] = plsc.load_gather(s4_v, [rowv, colv])
    pltpu.sync_copy(out_v, out_hbm.at[pl.ds(base * _NCAND, nc)])


_sc_select = functools.partial(
    pl.kernel,
    mesh=plsc.VectorSubcoreMesh(core_axis_name="c", subcore_axis_name="s"),
    out_type=jax.ShapeDtypeStruct((_B * _NCAND,), jnp.float32),
    scratch_types=[
        pltpu.VMEM((_ROWS_PER_W, _NL), jnp.float32),
        pltpu.VMEM((_ROWS_PER_W * _NCAND,), jnp.int32),
        pltpu.VMEM((_ROWS_PER_W * _NCAND,), jnp.float32),
    ],
    compiler_params=pltpu.CompilerParams(
        needs_layout_passes=False, use_tc_tiling_on_sc=False),
)(_sc_select_body)


@jax.jit
def kernel(hidden_states, labels, W1, b1, W2, b2, group_y, embed_table):
    del labels
    cls_feats = jnp.concatenate(
        [hidden_states[-i][:, 0] for i in range(1, _FEATURE_LAYERS + 1)],
        axis=-1)
    logits = _matmul_bias(cls_feats, W1, b1, 1024)
    cand = _sc_topk_route(logits, group_y.reshape(-1))
    embT = _matmulT_bias(W2.T, logits, b2, 512)
    scores = _score_all(embT.T, embed_table.T)
    return _sc_select(scores, cand).reshape(_B, _NCAND)
